# baseline (speedup 1.0000x reference)
import functools

import jax
import jax.numpy as jnp
from jax import lax
from jax.experimental import pallas as pl
from jax.experimental.pallas import tpu as pltpu
from jax.experimental.pallas import tpu_sc as plsc

BATCH = 16384
EMBED_DIM = 64
CHUNK = 128

_info = plsc.get_sparse_core_info()
NUM_CORES = _info.num_cores
NUM_SUBCORES = _info.num_subcores
NUM_WORKERS = NUM_CORES * NUM_SUBCORES
ROWS_PER_WORKER = BATCH // NUM_WORKERS
CHUNKS_PER_WORKER = ROWS_PER_WORKER // CHUNK


def _compute_chunk(h_v, r_v, t_v, o_v):
    def row_body(i, carry):
        for k in range(EMBED_DIM // 16):
            sl = pl.ds(k * 16, 16)
            o_v[i, sl] = jnp.abs(h_v[i, sl] + r_v[i, sl] - t_v[i, sl])
        return carry

    lax.fori_loop(0, CHUNK, row_body, 0, unroll=2)


def _process_set(ent_hbm, rel_hbm, ih_hbm, ir_hbm, it_hbm, out_hbm,
                 wid, ih_v, ir_v, it_v, h_v, r_v, t_v, o_v, sem):
    pltpu.sync_copy(ih_hbm.at[wid], ih_v)
    pltpu.sync_copy(ir_hbm.at[wid], ir_v)
    pltpu.sync_copy(it_hbm.at[wid], it_v)
    base = wid * ROWS_PER_WORKER

    def chunk_body(c, carry):
        cp_h = pltpu.async_copy(ent_hbm.at[ih_v.at[c]], h_v, sem)
        cp_r = pltpu.async_copy(rel_hbm.at[ir_v.at[c]], r_v, sem)
        cp_t = pltpu.async_copy(ent_hbm.at[it_v.at[c]], t_v, sem)
        cp_h.wait()
        cp_r.wait()
        cp_t.wait()
        _compute_chunk(h_v, r_v, t_v, o_v)
        pltpu.sync_copy(o_v, out_hbm.at[pl.ds(base + c * CHUNK, CHUNK)])
        return carry

    lax.fori_loop(0, CHUNKS_PER_WORKER, chunk_body, 0)


def _transe_kernel(ent_hbm, rel_hbm,
                   ph_hbm, pr_hbm, pt_hbm, nh_hbm, nr_hbm, nt_hbm,
                   pos_out, neg_out,
                   ih_v, ir_v, it_v, h_v, r_v, t_v, o_v, sem):
    wid = lax.axis_index("s") * NUM_CORES + lax.axis_index("c")
    _process_set(ent_hbm, rel_hbm, ph_hbm, pr_hbm, pt_hbm, pos_out,
                 wid, ih_v, ir_v, it_v, h_v, r_v, t_v, o_v, sem)
    _process_set(ent_hbm, rel_hbm, nh_hbm, nr_hbm, nt_hbm, neg_out,
                 wid, ih_v, ir_v, it_v, h_v, r_v, t_v, o_v, sem)


@jax.jit
def kernel(positive_samples, negative_samples, entity_embedding, relation_embedding):
    idx_shape = (NUM_WORKERS, CHUNKS_PER_WORKER, CHUNK)
    ph = positive_samples[:, 0].reshape(idx_shape)
    pr = positive_samples[:, 1].reshape(idx_shape)
    pt = positive_samples[:, 2].reshape(idx_shape)
    nh = negative_samples[:, 0].reshape(idx_shape)
    nr = negative_samples[:, 1].reshape(idx_shape)
    nt = negative_samples[:, 2].reshape(idx_shape)

    mesh = plsc.VectorSubcoreMesh(core_axis_name="c", subcore_axis_name="s")
    out_t = jax.ShapeDtypeStruct((BATCH, EMBED_DIM), jnp.float32)
    run = pl.kernel(
        _transe_kernel,
        out_type=(out_t, out_t),
        mesh=mesh,
        compiler_params=pltpu.CompilerParams(use_tc_tiling_on_sc=False),
        scratch_types=[
            pltpu.VMEM((CHUNKS_PER_WORKER, CHUNK), jnp.int32),
            pltpu.VMEM((CHUNKS_PER_WORKER, CHUNK), jnp.int32),
            pltpu.VMEM((CHUNKS_PER_WORKER, CHUNK), jnp.int32),
            pltpu.VMEM((CHUNK, EMBED_DIM), jnp.float32),
            pltpu.VMEM((CHUNK, EMBED_DIM), jnp.float32),
            pltpu.VMEM((CHUNK, EMBED_DIM), jnp.float32),
            pltpu.VMEM((CHUNK, EMBED_DIM), jnp.float32),
            pltpu.SemaphoreType.DMA,
        ],
    )
    pos_out, neg_out = run(entity_embedding, relation_embedding,
                           ph, pr, pt, nh, nr, nt)
    return pos_out, neg_out


# speedup vs baseline: 1.0169x; 1.0169x over previous
import jax
import jax.numpy as jnp
from jax import lax
from jax.experimental import pallas as pl
from jax.experimental.pallas import tpu as pltpu
from jax.experimental.pallas import tpu_sc as plsc

BATCH = 16384
EMBED_DIM = 64
CHUNK = 128
NBUF = 4

_info = plsc.get_sparse_core_info()
NUM_CORES = _info.num_cores
NUM_SUBCORES = _info.num_subcores
NUM_WORKERS = NUM_CORES * NUM_SUBCORES
ROWS_PER_WORKER = BATCH // NUM_WORKERS
SET_CHUNKS = ROWS_PER_WORKER // CHUNK
TOTAL_CHUNKS = 2 * SET_CHUNKS


def _transe_kernel(ent_hbm, rel_hbm,
                   ph_hbm, pr_hbm, pt_hbm, nh_hbm, nr_hbm, nt_hbm,
                   pos_out, neg_out,
                   ih_v, ir_v, it_v, h_v, r_v, t_v,
                   sem0, sem1, sem2, sem3):
    wid = lax.axis_index("s") * NUM_CORES + lax.axis_index("c")
    wbase = wid * ROWS_PER_WORKER
    sems = [sem0, sem1, sem2, sem3]

    pltpu.sync_copy(ph_hbm.at[wid], ih_v.at[pl.ds(0, SET_CHUNKS)])
    pltpu.sync_copy(pr_hbm.at[wid], ir_v.at[pl.ds(0, SET_CHUNKS)])
    pltpu.sync_copy(pt_hbm.at[wid], it_v.at[pl.ds(0, SET_CHUNKS)])
    pltpu.sync_copy(nh_hbm.at[wid], ih_v.at[pl.ds(SET_CHUNKS, SET_CHUNKS)])
    pltpu.sync_copy(nr_hbm.at[wid], ir_v.at[pl.ds(SET_CHUNKS, SET_CHUNKS)])
    pltpu.sync_copy(nt_hbm.at[wid], it_v.at[pl.ds(SET_CHUNKS, SET_CHUNKS)])

    def fire(g, s):
        sem = sems[s]
        sl = pl.ds(s * CHUNK, CHUNK)
        return (
            pltpu.async_copy(ent_hbm.at[ih_v.at[g]], h_v.at[sl], sem),
            pltpu.async_copy(rel_hbm.at[ir_v.at[g]], r_v.at[sl], sem),
            pltpu.async_copy(ent_hbm.at[it_v.at[g]], t_v.at[sl], sem),
        )

    inflight = {}
    for g in range(NBUF):
        inflight[g] = fire(g, g % NBUF)

    for g in range(TOTAL_CHUNKS):
        s = g % NBUF
        for cp in inflight.pop(g):
            cp.wait()
        base = s * CHUNK

        def row_body(i, carry):
            for k in range(EMBED_DIM // 16):
                sl = pl.ds(k * 16, 16)
                h_v[base + i, sl] = jnp.abs(
                    h_v[base + i, sl] + r_v[base + i, sl] - t_v[base + i, sl])
            return carry

        lax.fori_loop(0, CHUNK, row_body, 0, unroll=4)

        out_hbm = pos_out if g < SET_CHUNKS else neg_out
        row0 = wbase + (g % SET_CHUNKS) * CHUNK
        pltpu.sync_copy(h_v.at[pl.ds(base, CHUNK)],
                        out_hbm.at[pl.ds(row0, CHUNK)])
        if g + NBUF < TOTAL_CHUNKS:
            inflight[g + NBUF] = fire(g + NBUF, s)


@jax.jit
def kernel(positive_samples, negative_samples, entity_embedding, relation_embedding):
    idx_shape = (NUM_WORKERS, SET_CHUNKS, CHUNK)
    ph = positive_samples[:, 0].reshape(idx_shape)
    pr = positive_samples[:, 1].reshape(idx_shape)
    pt = positive_samples[:, 2].reshape(idx_shape)
    nh = negative_samples[:, 0].reshape(idx_shape)
    nr = negative_samples[:, 1].reshape(idx_shape)
    nt = negative_samples[:, 2].reshape(idx_shape)

    mesh = plsc.VectorSubcoreMesh(core_axis_name="c", subcore_axis_name="s")
    out_t = jax.ShapeDtypeStruct((BATCH, EMBED_DIM), jnp.float32)
    run = pl.kernel(
        _transe_kernel,
        out_type=(out_t, out_t),
        mesh=mesh,
        compiler_params=pltpu.CompilerParams(use_tc_tiling_on_sc=False),
        scratch_types=[
            pltpu.VMEM((TOTAL_CHUNKS, CHUNK), jnp.int32),
            pltpu.VMEM((TOTAL_CHUNKS, CHUNK), jnp.int32),
            pltpu.VMEM((TOTAL_CHUNKS, CHUNK), jnp.int32),
            pltpu.VMEM((NBUF * CHUNK, EMBED_DIM), jnp.float32),
            pltpu.VMEM((NBUF * CHUNK, EMBED_DIM), jnp.float32),
            pltpu.VMEM((NBUF * CHUNK, EMBED_DIM), jnp.float32),
            pltpu.SemaphoreType.DMA,
            pltpu.SemaphoreType.DMA,
            pltpu.SemaphoreType.DMA,
            pltpu.SemaphoreType.DMA,
        ],
    )
    pos_out, neg_out = run(entity_embedding, relation_embedding,
                           ph, pr, pt, nh, nr, nt)
    return pos_out, neg_out
